# baseline (device time: 43447 ns/iter reference)
import jax
import jax.numpy as jnp
from jax import lax
from jax.experimental import pallas as pl
from jax.experimental.pallas import tpu as pltpu

CHUNK = 128


def kernel(Q, K, V):
    b, sq, h, d = Q.shape
    _, skv, _, _ = K.shape
    hd = h * d
    scale = d ** -0.5
    nsteps = skv // CHUNK

    def body(q_ref, k_ref, v_ref, out_ref, accs, ls, send_buf, recv_buf,
             send_sem, recv_sem):
        i = pl.program_id(0)

        eye8 = (
            lax.broadcasted_iota(jnp.int32, (h, h), 0)
            == lax.broadcasted_iota(jnp.int32, (h, h), 1)
        ).astype(jnp.float32)

        for bi in range(b):
            q = q_ref[bi, 0]
            qblock = (q[:, :, None] * eye8[:, None, :]).reshape(hd, h)
            k2 = k_ref[bi].reshape(CHUNK, hd)
            v2 = v_ref[bi].reshape(CHUNK, hd)
            s = lax.dot_general(
                k2, qblock, (((1,), (0,)), ((), ())),
                preferred_element_type=jnp.float32,
            ) * scale
            p = jnp.exp(s)
            lb = jnp.sum(p, axis=0, keepdims=True)
            a2 = lax.dot_general(
                p, v2, (((0,), (0,)), ((), ())),
                preferred_element_type=jnp.float32,
            )

            @pl.when(i == 0)
            def _():
                accs[bi] = a2
                ls[bi] = lb

            @pl.when(i != 0)
            def _():
                accs[bi] = accs[bi] + a2
                ls[bi] = ls[bi] + lb

        @pl.when(i == nsteps - 1)
        def _():
            my_x = lax.axis_index("x")
            my_y = lax.axis_index("y")
            my_z = lax.axis_index("z")
            nbr = (1 - my_x, my_y, my_z)

            barrier = pltpu.get_barrier_semaphore()
            pl.semaphore_signal(barrier, inc=1, device_id=nbr,
                                device_id_type=pl.DeviceIdType.MESH)
            pl.semaphore_wait(barrier, 1)

            for bi in range(b):
                a3 = accs[bi].reshape(h, h, d)
                abh = jnp.sum(a3 * eye8[:, :, None], axis=1)
                send_buf[0, bi] = abh
                send_buf[1, bi] = jnp.broadcast_to(
                    ls[bi].reshape(h, 1), (h, d)
                )

            rdma = pltpu.make_async_remote_copy(
                src_ref=send_buf,
                dst_ref=recv_buf,
                send_sem=send_sem,
                recv_sem=recv_sem,
                device_id=nbr,
                device_id_type=pl.DeviceIdType.MESH,
            )
            rdma.start()
            rdma.wait()

            acc = send_buf[0] + recv_buf[0]
            l_all = send_buf[1] + recv_buf[1]
            out_ref[...] = (acc / l_all).reshape(b, sq, h, d)

    return pl.pallas_call(
        body,
        grid=(nsteps,),
        out_shape=jax.ShapeDtypeStruct((b, sq, h, d), jnp.float32),
        in_specs=[
            pl.BlockSpec((b, sq, h, d), lambda i: (0, 0, 0, 0),
                         memory_space=pltpu.VMEM),
            pl.BlockSpec((b, CHUNK, h, d), lambda i: (0, i, 0, 0),
                         memory_space=pltpu.VMEM),
            pl.BlockSpec((b, CHUNK, h, d), lambda i: (0, i, 0, 0),
                         memory_space=pltpu.VMEM),
        ],
        out_specs=pl.BlockSpec((b, sq, h, d), lambda i: (0, 0, 0, 0),
                               memory_space=pltpu.VMEM),
        scratch_shapes=[
            pltpu.VMEM((b, h, hd), jnp.float32),
            pltpu.VMEM((b, 1, h), jnp.float32),
            pltpu.VMEM((2, b, h, d), jnp.float32),
            pltpu.VMEM((2, b, h, d), jnp.float32),
            pltpu.SemaphoreType.DMA,
            pltpu.SemaphoreType.DMA,
        ],
        compiler_params=pltpu.CompilerParams(collective_id=0),
    )(Q, K, V)


# device time: 22908 ns/iter; 1.8966x vs baseline; 1.8966x over previous
import jax
import jax.numpy as jnp
from jax import lax
from jax.experimental import pallas as pl
from jax.experimental.pallas import tpu as pltpu

CHUNK = 128


def kernel(Q, K, V):
    b, sq, h, d = Q.shape
    _, skv, _, _ = K.shape
    hd = h * d
    bh = b * h
    bk = b * CHUNK
    scale = d ** -0.5
    nsteps = skv // CHUNK

    K2 = K.reshape(b, skv, hd)
    V2 = V.reshape(b, skv, hd)

    def body(q_ref, k_ref, v_ref, out_ref, q2t_ref, acc_ref, l_ref,
             send_buf, recv_buf, send_sem, recv_sem):
        i = pl.program_id(0)

        eye8 = (
            lax.broadcasted_iota(jnp.int32, (h, h), 0)
            == lax.broadcasted_iota(jnp.int32, (h, h), 1)
        ).astype(jnp.float32)

        my_x = lax.axis_index("x")
        my_y = lax.axis_index("y")
        my_z = lax.axis_index("z")
        nbr = (1 - my_x, my_y, my_z)
        barrier = pltpu.get_barrier_semaphore()

        @pl.when(i == 0)
        def _():
            pl.semaphore_signal(barrier, inc=1, device_id=nbr,
                                device_id_type=pl.DeviceIdType.MESH)
            for bi in range(b):
                q = q_ref[bi, 0]
                q2t_ref[bi] = (
                    eye8[:, :, None] * q[:, None, :]
                ).reshape(h, hd)

        with jax.named_scope("dot_s"):
            k3 = k_ref[...].reshape(bk, hd)
            q2t = q2t_ref[...].reshape(bh, hd)
            s = lax.dot_general(
                k3, q2t, (((1,), (1,)), ((), ())),
                preferred_element_type=jnp.float32,
            ) * scale

        with jax.named_scope("exp_mask"):
            rb = lax.broadcasted_iota(jnp.int32, (bk, bh), 0) // CHUNK
            cb = lax.broadcasted_iota(jnp.int32, (bk, bh), 1) // h
            mask = (rb == cb).astype(jnp.float32)
            p = jnp.exp(s) * mask
            lb = lax.dot_general(
                p, jnp.ones((bk, d), jnp.float32), (((0,), (0,)), ((), ())),
                preferred_element_type=jnp.float32,
            )

        with jax.named_scope("dot_av"):
            v3 = v_ref[...].reshape(bk, hd)
            a = lax.dot_general(
                p, v3, (((0,), (0,)), ((), ())),
                preferred_element_type=jnp.float32,
            )

        @pl.when(i == 0)
        def _():
            acc_ref[...] = a
            l_ref[...] = lb

        @pl.when(i != 0)
        def _():
            acc_ref[...] = acc_ref[...] + a
            l_ref[...] = l_ref[...] + lb

        @pl.when(i == nsteps - 1)
        def _():
            with jax.named_scope("pack"):
                a4 = acc_ref[...].reshape(b, h, h, d)
                abh = jnp.sum(a4 * eye8[None, :, :, None], axis=2)
                send_buf[0] = abh
                send_buf[1] = l_ref[...].reshape(b, h, d)

            with jax.named_scope("exchange"):
                pl.semaphore_wait(barrier, 1)
                rdma = pltpu.make_async_remote_copy(
                    src_ref=send_buf,
                    dst_ref=recv_buf,
                    send_sem=send_sem,
                    recv_sem=recv_sem,
                    device_id=nbr,
                    device_id_type=pl.DeviceIdType.MESH,
                )
                rdma.start()
                rdma.wait()

            with jax.named_scope("combine"):
                acc = send_buf[0] + recv_buf[0]
                l_all = send_buf[1] + recv_buf[1]
                out_ref[...] = (acc / l_all).reshape(b, sq, h, d)

    return pl.pallas_call(
        body,
        grid=(nsteps,),
        out_shape=jax.ShapeDtypeStruct((b, sq, h, d), jnp.float32),
        in_specs=[
            pl.BlockSpec((b, sq, h, d), lambda i: (0, 0, 0, 0),
                         memory_space=pltpu.VMEM),
            pl.BlockSpec((b, CHUNK, hd), lambda i: (0, i, 0),
                         memory_space=pltpu.VMEM),
            pl.BlockSpec((b, CHUNK, hd), lambda i: (0, i, 0),
                         memory_space=pltpu.VMEM),
        ],
        out_specs=pl.BlockSpec((b, sq, h, d), lambda i: (0, 0, 0, 0),
                               memory_space=pltpu.VMEM),
        scratch_shapes=[
            pltpu.VMEM((b, h, hd), jnp.float32),
            pltpu.VMEM((bh, hd), jnp.float32),
            pltpu.VMEM((bh, d), jnp.float32),
            pltpu.VMEM((2, b, h, d), jnp.float32),
            pltpu.VMEM((2, b, h, d), jnp.float32),
            pltpu.SemaphoreType.DMA,
            pltpu.SemaphoreType.DMA,
        ],
        compiler_params=pltpu.CompilerParams(collective_id=0),
    )(Q, K2, V2)


# device time: 17913 ns/iter; 2.4254x vs baseline; 1.2788x over previous
import jax
import jax.numpy as jnp
from jax import lax
from jax.experimental import pallas as pl
from jax.experimental.pallas import tpu as pltpu


def kernel(Q, K, V):
    b, sq, h, d = Q.shape
    _, skv, _, _ = K.shape
    hd = h * d
    scale = d ** -0.5

    KT = K.transpose(0, 2, 3, 1).reshape(b, hd, skv)
    VT = V.transpose(0, 2, 3, 1).reshape(b, hd, skv)

    def body(q_ref, k_ref, v_ref, out_ref, send_buf, recv_buf,
             send_sem, recv_sem):
        bi = pl.program_id(0)

        eye8 = (
            lax.broadcasted_iota(jnp.int32, (h, h), 0)
            == lax.broadcasted_iota(jnp.int32, (h, h), 1)
        ).astype(jnp.float32)

        my_x = lax.axis_index("x")
        my_y = lax.axis_index("y")
        my_z = lax.axis_index("z")
        nbr = (1 - my_x, my_y, my_z)
        barrier = pltpu.get_barrier_semaphore()

        @pl.when(bi == 0)
        def _():
            pl.semaphore_signal(barrier, inc=1, device_id=nbr,
                                device_id_type=pl.DeviceIdType.MESH)

        with jax.named_scope("dot_s"):
            q = q_ref[bi, 0]
            qbt = (eye8[:, :, None] * q[:, None, :]).reshape(h, hd)
            kt = k_ref[0]
            st = lax.dot_general(
                qbt, kt, (((1,), (0,)), ((), ())),
                preferred_element_type=jnp.float32,
            ) * scale
            p = jnp.exp(st)

        with jax.named_scope("dot_av"):
            vt = v_ref[0]
            a = lax.dot_general(
                p, vt, (((1,), (1,)), ((), ())),
                preferred_element_type=jnp.float32,
            )
            lb = lax.dot_general(
                p, jnp.ones((skv, d), jnp.float32), (((1,), (0,)), ((), ())),
                preferred_element_type=jnp.float32,
            )

        with jax.named_scope("pack"):
            a3 = a.reshape(h, h, d)
            abh = jnp.sum(a3 * eye8[:, :, None], axis=1)
            send_buf[0, bi] = abh
            send_buf[1, bi] = lb

        @pl.when(bi == b - 1)
        def _():
            with jax.named_scope("exchange"):
                pl.semaphore_wait(barrier, 1)
                rdma = pltpu.make_async_remote_copy(
                    src_ref=send_buf,
                    dst_ref=recv_buf,
                    send_sem=send_sem,
                    recv_sem=recv_sem,
                    device_id=nbr,
                    device_id_type=pl.DeviceIdType.MESH,
                )
                rdma.start()
                rdma.wait()

            with jax.named_scope("combine"):
                acc = send_buf[0] + recv_buf[0]
                l_all = send_buf[1] + recv_buf[1]
                out_ref[...] = (acc / l_all).reshape(b, sq, h, d)

    return pl.pallas_call(
        body,
        grid=(b,),
        out_shape=jax.ShapeDtypeStruct((b, sq, h, d), jnp.float32),
        in_specs=[
            pl.BlockSpec((b, sq, h, d), lambda i: (0, 0, 0, 0),
                         memory_space=pltpu.VMEM),
            pl.BlockSpec((1, hd, skv), lambda i: (i, 0, 0),
                         memory_space=pltpu.VMEM),
            pl.BlockSpec((1, hd, skv), lambda i: (i, 0, 0),
                         memory_space=pltpu.VMEM),
        ],
        out_specs=pl.BlockSpec((b, sq, h, d), lambda i: (0, 0, 0, 0),
                               memory_space=pltpu.VMEM),
        scratch_shapes=[
            pltpu.VMEM((2, b, h, d), jnp.float32),
            pltpu.VMEM((2, b, h, d), jnp.float32),
            pltpu.SemaphoreType.DMA,
            pltpu.SemaphoreType.DMA,
        ],
        compiler_params=pltpu.CompilerParams(collective_id=0),
    )(Q, KT, VT)
